# Initial kernel scaffold; baseline (speedup 1.0000x reference)
import jax
import jax.numpy as jnp
from jax.experimental import pallas as pl


def kernel(x, edge_index, W1l, b1, W1r, W2l, b2, W2r):
    raise NotImplementedError("write your pallas kernel here")


# speedup vs baseline: 2.6719x; 2.6719x over previous
import jax
import jax.numpy as jnp
from jax import lax
from jax.experimental import pallas as pl
from jax.experimental.pallas import tpu as pltpu
from jax.experimental.pallas import tpu_sc as plsc

N_NODES = 10000
N_EDGES = 320000
D = 128

NC = 2
NS = 16
NW = NC * NS
C = 128
K = 80
EPW = K * C
E_PAD = NW * EPW
NPAD = 10240
DUMMY = 10200
RPT = NPAD // NS


def _make_sc_agg(with_cnt):
    mesh = plsc.VectorSubcoreMesh(core_axis_name="c", subcore_axis_name="s")

    out_type = [jax.ShapeDtypeStruct((NC, NPAD, D), jnp.float32)]
    scratch = [
        pltpu.VMEM((1, C), jnp.int32),
        pltpu.VMEM((1, C), jnp.int32),
        pltpu.VMEM((C, D), jnp.float32),
        pltpu.VMEM_SHARED((NPAD, D), jnp.float32),
        pltpu.SemaphoreType.DMA,
    ]
    if with_cnt:
        out_type.append(jax.ShapeDtypeStruct((NC * NPAD,), jnp.float32))
        scratch += [
            pltpu.VMEM((RPT,), jnp.float32),
            pltpu.VMEM((C,), jnp.float32),
            pltpu.VMEM_SHARED((NPAD,), jnp.float32)
        ]

    def body(*refs):
        if with_cnt:
            (table_hbm, src_hbm, dst_hbm, agg_out, cnt_out,
             sidx, didx, rows, aggS, sem, cntV, onesC, cntS) = refs
        else:
            (table_hbm, src_hbm, dst_hbm, agg_out,
             sidx, didx, rows, aggS, sem) = refs

        cid = lax.axis_index("c")
        sid = lax.axis_index("s")
        wid = sid * NC + cid

        def zrow(i, _):
            r = i // (D // 16)
            c = lax.rem(i, D // 16)
            rows[r, pl.ds(c * 16, 16)] = jnp.zeros((16,), jnp.float32)
            return 0
        lax.fori_loop(0, C * (D // 16), zrow, 0)

        base = sid * RPT
        nfull = RPT // C
        for j in range(nfull):
            pltpu.sync_copy(rows, aggS.at[pl.ds(base + j * C, C)])

        if with_cnt:
            def zcnt(i, _):
                cntV[pl.ds(i * 16, 16)] = jnp.zeros((16,), jnp.float32)
                return 0
            lax.fori_loop(0, RPT // 16, zcnt, 0)
            pltpu.sync_copy(cntV, cntS.at[pl.ds(base, RPT)])

            def ofill(i, _):
                onesC[pl.ds(i * 16, 16)] = jnp.ones((16,), jnp.float32)
                return 0
            lax.fori_loop(0, C // 16, ofill, 0)

        plsc.subcore_barrier()

        def step(c, _):
            pltpu.sync_copy(src_hbm.at[wid, c], sidx)
            pltpu.sync_copy(dst_hbm.at[wid, c], didx)
            pltpu.async_copy(table_hbm.at[sidx.at[0]], rows, sem).wait()
            pltpu.sync_copy(rows, aggS.at[didx.at[0]], add=True)
            if with_cnt:
                pltpu.sync_copy(onesC, cntS.at[didx.at[0]], add=True)
            return 0
        lax.fori_loop(0, K, step, 0)

        plsc.subcore_barrier()

        if with_cnt:
            pltpu.sync_copy(cntS.at[pl.ds(base, RPT)], cntV)
            pltpu.sync_copy(cntV,
                            cnt_out.at[pl.ds(cid * NPAD + base, RPT)])

        for j in range(nfull):
            pltpu.sync_copy(aggS.at[pl.ds(base + j * C, C)], rows)
            pltpu.sync_copy(rows, agg_out.at[cid, pl.ds(base + j * C, C)])

    out = tuple(out_type) if with_cnt else out_type[0]
    return pl.kernel(body, out_type=out, mesh=mesh,
                     scratch_types=tuple(scratch))


_sc_agg_l1 = _make_sc_agg(with_cnt=True)
_sc_agg_l2 = _make_sc_agg(with_cnt=False)

BR = 512
NBLK = NPAD // BR


def _tc1_body(agg_ref, cnt_ref, x_ref, wl_ref, b_ref, wr_ref,
              h_ref, inv_ref):
    onesw = jnp.ones((NC, 1), jnp.float32)
    cnt = lax.dot_general(cnt_ref[...], onesw, (((0,), (0,)), ((), ())),
                          preferred_element_type=jnp.float32)
    inv = 1.0 / jnp.maximum(cnt, 1.0)
    a = (agg_ref[0] + agg_ref[1]) * inv
    hl = lax.dot_general(a, wl_ref[...], (((1,), (1,)), ((), ())),
                         preferred_element_type=jnp.float32)
    hr = lax.dot_general(x_ref[...], wr_ref[...], (((1,), (1,)), ((), ())),
                         preferred_element_type=jnp.float32)
    h_ref[...] = jnp.maximum(hl + hr + b_ref[...], 0.0)
    inv_ref[...] = jnp.broadcast_to(inv, (BR, 16))


def _tc2_body(agg_ref, inv_ref, h_ref, wl_ref, b_ref, wr_ref, out_ref):
    inv = inv_ref[:, 0:1]
    a = (agg_ref[0] + agg_ref[1]) * inv
    ol = lax.dot_general(a, wl_ref[...], (((1,), (1,)), ((), ())),
                         preferred_element_type=jnp.float32)
    orr = lax.dot_general(h_ref[...], wr_ref[...], (((1,), (1,)), ((), ())),
                          preferred_element_type=jnp.float32)
    out_ref[...] = ol + orr + b_ref[...]


_tc1 = pl.pallas_call(
    _tc1_body,
    grid=(NBLK,),
    in_specs=[
        pl.BlockSpec((NC, BR, D), lambda i: (0, i, 0)),
        pl.BlockSpec((NC, BR), lambda i: (0, i)),
        pl.BlockSpec((BR, D), lambda i: (i, 0)),
        pl.BlockSpec((D, D), lambda i: (0, 0)),
        pl.BlockSpec((1, D), lambda i: (0, 0)),
        pl.BlockSpec((D, D), lambda i: (0, 0)),
    ],
    out_specs=[
        pl.BlockSpec((BR, D), lambda i: (i, 0)),
        pl.BlockSpec((BR, 16), lambda i: (i, 0)),
    ],
    out_shape=[
        jax.ShapeDtypeStruct((NPAD, D), jnp.float32),
        jax.ShapeDtypeStruct((NPAD, 16), jnp.float32),
    ],
)

_tc2 = pl.pallas_call(
    _tc2_body,
    grid=(NBLK,),
    in_specs=[
        pl.BlockSpec((NC, BR, D), lambda i: (0, i, 0)),
        pl.BlockSpec((BR, 16), lambda i: (i, 0)),
        pl.BlockSpec((BR, D), lambda i: (i, 0)),
        pl.BlockSpec((D, D), lambda i: (0, 0)),
        pl.BlockSpec((1, D), lambda i: (0, 0)),
        pl.BlockSpec((D, D), lambda i: (0, 0)),
    ],
    out_specs=pl.BlockSpec((BR, D), lambda i: (i, 0)),
    out_shape=jax.ShapeDtypeStruct((NPAD, D), jnp.float32),
)


def kernel(x, edge_index, W1l, b1, W1r, W2l, b2, W2r):
    ei = edge_index.astype(jnp.int32)
    pad = E_PAD - N_EDGES
    src = jnp.concatenate([ei[0], jnp.zeros((pad,), jnp.int32)])
    dst = jnp.concatenate([ei[1], jnp.full((pad,), DUMMY, jnp.int32)])
    src = src.reshape(NW, K, 1, C)
    dst = dst.reshape(NW, K, 1, C)

    xp = jnp.concatenate(
        [x, jnp.zeros((NPAD - N_NODES, D), jnp.float32)], axis=0)
    agg1p, cnt1p = _sc_agg_l1(xp, src, dst)
    h, inv = _tc1(agg1p, cnt1p.reshape(NC, NPAD), xp,
                  W1l, b1.reshape(1, D), W1r)
    agg2p = _sc_agg_l2(h, src, dst)
    out = _tc2(agg2p, inv, h, W2l, b2.reshape(1, D), W2r)
    return out[:N_NODES]


# speedup vs baseline: 3.2749x; 1.2257x over previous
import jax
import jax.numpy as jnp
from jax import lax
from jax.experimental import pallas as pl
from jax.experimental.pallas import tpu as pltpu
from jax.experimental.pallas import tpu_sc as plsc

N_NODES = 10000
N_EDGES = 320000
D = 128

NC = 2
NS = 16
NW = NC * NS
C = 128
K = 80
EPW = K * C
E_PAD = NW * EPW
NPAD = 10240
DUMMY = 10200
RPT = NPAD // NS


def _make_sc_agg(with_cnt):
    mesh = plsc.VectorSubcoreMesh(core_axis_name="c", subcore_axis_name="s")

    out_type = [jax.ShapeDtypeStruct((NC, NPAD, D), jnp.float32)]
    scratch = [
        pltpu.VMEM((1, C), jnp.int32),
        pltpu.VMEM((1, C), jnp.int32),
        pltpu.VMEM((1, C), jnp.int32),
        pltpu.VMEM((1, C), jnp.int32),
        pltpu.VMEM((C, D), jnp.float32),
        pltpu.VMEM((C, D), jnp.float32),
        pltpu.VMEM_SHARED((NPAD, D), jnp.float32),
        pltpu.SemaphoreType.DMA,
        pltpu.SemaphoreType.DMA,
    ]
    if with_cnt:
        out_type.append(jax.ShapeDtypeStruct((NC * NPAD,), jnp.float32))
        scratch += [
            pltpu.VMEM((RPT,), jnp.float32),
            pltpu.VMEM((C,), jnp.float32),
            pltpu.VMEM_SHARED((NPAD,), jnp.float32)
        ]

    def body(*refs):
        if with_cnt:
            (table_hbm, src_hbm, dst_hbm, agg_out, cnt_out,
             sidx0, didx0, sidx1, didx1, rows0, rows1, aggS, sem0, sem1,
             cntV, onesC, cntS) = refs
        else:
            (table_hbm, src_hbm, dst_hbm, agg_out,
             sidx0, didx0, sidx1, didx1, rows0, rows1, aggS,
             sem0, sem1) = refs
        sidx = (sidx0, sidx1)
        didx = (didx0, didx1)
        rows_b = (rows0, rows1)
        sem = (sem0, sem1)
        rows = rows0

        cid = lax.axis_index("c")
        sid = lax.axis_index("s")
        wid = sid * NC + cid

        def zrow(i, _):
            r = i // (D // 16)
            c = lax.rem(i, D // 16)
            rows[r, pl.ds(c * 16, 16)] = jnp.zeros((16,), jnp.float32)
            return 0
        lax.fori_loop(0, C * (D // 16), zrow, 0)

        base = sid * RPT
        nfull = RPT // C
        for j in range(nfull):
            pltpu.sync_copy(rows, aggS.at[pl.ds(base + j * C, C)])

        if with_cnt:
            def zcnt(i, _):
                cntV[pl.ds(i * 16, 16)] = jnp.zeros((16,), jnp.float32)
                return 0
            lax.fori_loop(0, RPT // 16, zcnt, 0)
            pltpu.sync_copy(cntV, cntS.at[pl.ds(base, RPT)])

            def ofill(i, _):
                onesC[pl.ds(i * 16, 16)] = jnp.ones((16,), jnp.float32)
                return 0
            lax.fori_loop(0, C // 16, ofill, 0)

        pltpu.sync_copy(src_hbm.at[wid, 0], sidx[0])
        pltpu.sync_copy(dst_hbm.at[wid, 0], didx[0])
        pltpu.async_copy(table_hbm.at[sidx[0].at[0]], rows_b[0], sem[0])

        plsc.subcore_barrier()

        def step2(g, _):
            for p in (0, 1):
                i = 2 * g + p
                nb = 1 - p

                @pl.when(i + 1 < K)
                def _():
                    pltpu.sync_copy(src_hbm.at[wid, i + 1], sidx[nb])
                    pltpu.sync_copy(dst_hbm.at[wid, i + 1], didx[nb])
                    pltpu.async_copy(table_hbm.at[sidx[nb].at[0]],
                                     rows_b[nb], sem[nb])

                pltpu.make_async_copy(table_hbm.at[sidx[p].at[0]],
                                      rows_b[p], sem[p]).wait()
                pltpu.sync_copy(rows_b[p], aggS.at[didx[p].at[0]], add=True)
                if with_cnt:
                    pltpu.sync_copy(onesC, cntS.at[didx[p].at[0]], add=True)
            return 0
        lax.fori_loop(0, K // 2, step2, 0)

        plsc.subcore_barrier()

        if with_cnt:
            pltpu.sync_copy(cntS.at[pl.ds(base, RPT)], cntV)
            pltpu.sync_copy(cntV,
                            cnt_out.at[pl.ds(cid * NPAD + base, RPT)])

        for j in range(nfull):
            pltpu.sync_copy(aggS.at[pl.ds(base + j * C, C)], rows)
            pltpu.sync_copy(rows, agg_out.at[cid, pl.ds(base + j * C, C)])

    out = tuple(out_type) if with_cnt else out_type[0]
    return pl.kernel(body, out_type=out, mesh=mesh,
                     scratch_types=tuple(scratch))


_sc_agg_l1 = _make_sc_agg(with_cnt=True)
_sc_agg_l2 = _make_sc_agg(with_cnt=False)

BR = 512
NBLK = NPAD // BR


def _tc1_body(agg_ref, cnt_ref, x_ref, wl_ref, b_ref, wr_ref,
              h_ref, inv_ref):
    onesw = jnp.ones((NC, 1), jnp.float32)
    cnt = lax.dot_general(cnt_ref[...], onesw, (((0,), (0,)), ((), ())),
                          preferred_element_type=jnp.float32)
    inv = 1.0 / jnp.maximum(cnt, 1.0)
    a = (agg_ref[0] + agg_ref[1]) * inv
    hl = lax.dot_general(a, wl_ref[...], (((1,), (1,)), ((), ())),
                         preferred_element_type=jnp.float32)
    hr = lax.dot_general(x_ref[...], wr_ref[...], (((1,), (1,)), ((), ())),
                         preferred_element_type=jnp.float32)
    h_ref[...] = jnp.maximum(hl + hr + b_ref[...], 0.0)
    inv_ref[...] = jnp.broadcast_to(inv, (BR, 16))


def _tc2_body(agg_ref, inv_ref, h_ref, wl_ref, b_ref, wr_ref, out_ref):
    inv = inv_ref[:, 0:1]
    a = (agg_ref[0] + agg_ref[1]) * inv
    ol = lax.dot_general(a, wl_ref[...], (((1,), (1,)), ((), ())),
                         preferred_element_type=jnp.float32)
    orr = lax.dot_general(h_ref[...], wr_ref[...], (((1,), (1,)), ((), ())),
                          preferred_element_type=jnp.float32)
    out_ref[...] = ol + orr + b_ref[...]


_tc1 = pl.pallas_call(
    _tc1_body,
    grid=(NBLK,),
    in_specs=[
        pl.BlockSpec((NC, BR, D), lambda i: (0, i, 0)),
        pl.BlockSpec((NC, BR), lambda i: (0, i)),
        pl.BlockSpec((BR, D), lambda i: (i, 0)),
        pl.BlockSpec((D, D), lambda i: (0, 0)),
        pl.BlockSpec((1, D), lambda i: (0, 0)),
        pl.BlockSpec((D, D), lambda i: (0, 0)),
    ],
    out_specs=[
        pl.BlockSpec((BR, D), lambda i: (i, 0)),
        pl.BlockSpec((BR, 16), lambda i: (i, 0)),
    ],
    out_shape=[
        jax.ShapeDtypeStruct((NPAD, D), jnp.float32),
        jax.ShapeDtypeStruct((NPAD, 16), jnp.float32),
    ],
)

_tc2 = pl.pallas_call(
    _tc2_body,
    grid=(NBLK,),
    in_specs=[
        pl.BlockSpec((NC, BR, D), lambda i: (0, i, 0)),
        pl.BlockSpec((BR, 16), lambda i: (i, 0)),
        pl.BlockSpec((BR, D), lambda i: (i, 0)),
        pl.BlockSpec((D, D), lambda i: (0, 0)),
        pl.BlockSpec((1, D), lambda i: (0, 0)),
        pl.BlockSpec((D, D), lambda i: (0, 0)),
    ],
    out_specs=pl.BlockSpec((BR, D), lambda i: (i, 0)),
    out_shape=jax.ShapeDtypeStruct((NPAD, D), jnp.float32),
)


def kernel(x, edge_index, W1l, b1, W1r, W2l, b2, W2r):
    ei = edge_index.astype(jnp.int32)
    pad = E_PAD - N_EDGES
    src = jnp.concatenate([ei[0], jnp.zeros((pad,), jnp.int32)])
    dst = jnp.concatenate([ei[1], jnp.full((pad,), DUMMY, jnp.int32)])
    src = src.reshape(NW, K, 1, C)
    dst = dst.reshape(NW, K, 1, C)

    xp = jnp.concatenate(
        [x, jnp.zeros((NPAD - N_NODES, D), jnp.float32)], axis=0)
    agg1p, cnt1p = _sc_agg_l1(xp, src, dst)
    h, inv = _tc1(agg1p, cnt1p.reshape(NC, NPAD), xp,
                  W1l, b1.reshape(1, D), W1r)
    agg2p = _sc_agg_l2(h, src, dst)
    out = _tc2(agg2p, inv, h, W2l, b2.reshape(1, D), W2r)
    return out[:N_NODES]
